# baseline (speedup 1.0000x reference)
import functools

import jax
import jax.numpy as jnp
from jax import lax
from jax.experimental import pallas as pl
from jax.experimental.pallas import tpu as pltpu
from jax.experimental.pallas import tpu_sc as plsc

EMB = 1_000_000
H1 = 32
HIST = 50
BATCH = 16384
H2 = HIST * H1
NCLS = 3

NC = 2
NS = 16
NW = NC * NS
TOTAL = BATCH * HIST
PER_W = TOTAL // NW
CHUNK = 128
N_CHUNK = PER_W // CHUNK

_mesh = plsc.VectorSubcoreMesh(core_axis_name="c", subcore_axis_name="s")

CT = 8192


def _tp_body(in_ref, o_ref):
    o_ref[:, 0:H1] = in_ref[...].T.astype(jnp.bfloat16)


def _tc_transpose_pad(tbl_t):
    return pl.pallas_call(
        _tp_body,
        grid=((EMB + CT - 1) // CT,),
        in_specs=[pl.BlockSpec((H1, CT), lambda j: (0, j))],
        out_specs=pl.BlockSpec((CT, 128), lambda j: (j, 0)),
        out_shape=jax.ShapeDtypeStruct((EMB, 128), jnp.bfloat16),
    )(tbl_t)


@functools.partial(
    pl.kernel,
    mesh=_mesh,
    out_type=jax.ShapeDtypeStruct((TOTAL, H1), jnp.bfloat16),
    compiler_params=pltpu.CompilerParams(use_tc_tiling_on_sc=False),
    scratch_types=[
        pltpu.VMEM((CHUNK,), jnp.int32),
        pltpu.VMEM((CHUNK, H1), jnp.bfloat16),
        pltpu.SemaphoreType.DMA,
    ],
)
def _sc_gather(idx_hbm, table_hbm, out_hbm, idx_v, rows_v, sem):
    wid = lax.axis_index("s") * NC + lax.axis_index("c")
    base = wid * PER_W

    @pl.loop(0, N_CHUNK)
    def _(i):
        off = base + i * CHUNK
        pltpu.sync_copy(idx_hbm.at[pl.ds(off, CHUNK)], idx_v)
        pltpu.async_copy(table_hbm.at[idx_v], rows_v, sem).wait()
        pltpu.sync_copy(rows_v, out_hbm.at[pl.ds(off, CHUNK)])


def _mm_body(g_ref, w_ref, b_ref, o_ref):
    logits = jnp.dot(g_ref[...], w_ref[...],
                     preferred_element_type=jnp.float32) + b_ref[...]
    m = jnp.max(logits, axis=1, keepdims=True)
    s = logits - m
    lse = jnp.log(jnp.sum(jnp.exp(s), axis=1, keepdims=True))
    o_ref[...] = s - lse


BB = 1024


def _tc_head(g, wt, b2):
    return pl.pallas_call(
        _mm_body,
        grid=(BATCH // BB,),
        in_specs=[
            pl.BlockSpec((BB, H2), lambda i: (i, 0)),
            pl.BlockSpec((H2, NCLS), lambda i: (0, 0)),
            pl.BlockSpec((1, NCLS), lambda i: (0, 0)),
        ],
        out_specs=pl.BlockSpec((BB, NCLS), lambda i: (i, 0)),
        out_shape=jax.ShapeDtypeStruct((BATCH, NCLS), jnp.float32),
    )(g, wt, b2)


def kernel(x, emb_table, fc_w, fc_b):
    xf = x.reshape(-1).astype(jnp.int32) * 4
    tp = _tc_transpose_pad(emb_table.T)
    t4 = tp.reshape(4 * EMB, H1)
    g = _sc_gather(xf, t4)
    g2 = g.reshape(BATCH, H2)
    return _tc_head(g2, fc_w.T.astype(jnp.bfloat16), fc_b.reshape(1, NCLS))


# speedup vs baseline: 2.1543x; 2.1543x over previous
import functools

import jax
import jax.numpy as jnp
from jax import lax
from jax.experimental import pallas as pl
from jax.experimental.pallas import tpu as pltpu
from jax.experimental.pallas import tpu_sc as plsc

EMB = 1_000_000
H1 = 32
HIST = 50
BATCH = 16384
H2 = HIST * H1
NCLS = 3

NC = 2
NS = 16
NW = NC * NS
TOTAL = BATCH * HIST
PER_W = TOTAL // NW
CHUNK = 128
N_CHUNK = PER_W // CHUNK

_mesh = plsc.VectorSubcoreMesh(core_axis_name="c", subcore_axis_name="s")

CT = 8192


def _tp_body(in_ref, o_ref):
    t = in_ref[...].T
    q = CT // 4
    for k in range(4):
        o_ref[:, 32 * k:32 * k + 32] = t[k * q:(k + 1) * q, :]


def _tc_transpose_pad(tbl_t):
    return pl.pallas_call(
        _tp_body,
        grid=((EMB + CT - 1) // CT,),
        in_specs=[pl.BlockSpec((H1, CT), lambda j: (0, j))],
        out_specs=pl.BlockSpec((CT // 4, 128), lambda j: (j, 0)),
        out_shape=jax.ShapeDtypeStruct((EMB // 4, 128), jnp.float32),
    )(tbl_t)


@functools.partial(
    pl.kernel,
    mesh=_mesh,
    out_type=jax.ShapeDtypeStruct((TOTAL, H1), jnp.float32),
    compiler_params=pltpu.CompilerParams(use_tc_tiling_on_sc=False),
    scratch_types=[
        pltpu.VMEM((CHUNK,), jnp.int32),
        pltpu.VMEM((CHUNK, H1), jnp.float32),
        pltpu.SemaphoreType.DMA,
    ],
)
def _sc_gather(idx_hbm, table_hbm, out_hbm, idx_v, rows_v, sem):
    wid = lax.axis_index("s") * NC + lax.axis_index("c")
    base = wid * PER_W

    @pl.loop(0, N_CHUNK)
    def _(i):
        off = base + i * CHUNK
        pltpu.sync_copy(idx_hbm.at[pl.ds(off, CHUNK)], idx_v)
        pltpu.async_copy(table_hbm.at[idx_v], rows_v, sem).wait()
        pltpu.sync_copy(rows_v, out_hbm.at[pl.ds(off, CHUNK)])


def _mm_body(g_ref, w_ref, b_ref, o_ref):
    logits = jnp.dot(g_ref[...], w_ref[...],
                     preferred_element_type=jnp.float32) + b_ref[...]
    m = jnp.max(logits, axis=1, keepdims=True)
    s = logits - m
    lse = jnp.log(jnp.sum(jnp.exp(s), axis=1, keepdims=True))
    o_ref[...] = s - lse


BB = 1024


def _tc_head(g, wt, b2):
    return pl.pallas_call(
        _mm_body,
        grid=(BATCH // BB,),
        in_specs=[
            pl.BlockSpec((BB, H2), lambda i: (i, 0)),
            pl.BlockSpec((H2, NCLS), lambda i: (0, 0)),
            pl.BlockSpec((1, NCLS), lambda i: (0, 0)),
        ],
        out_specs=pl.BlockSpec((BB, NCLS), lambda i: (i, 0)),
        out_shape=jax.ShapeDtypeStruct((BATCH, NCLS), jnp.float32),
    )(g, wt, b2)


def kernel(x, emb_table, fc_w, fc_b):
    xi = x.reshape(-1).astype(jnp.int32)
    xf = (xi & ~(CT - 1)) | ((xi & (CT // 4 - 1)) << 2) | ((xi >> 11) & 3)
    tp = _tc_transpose_pad(emb_table.T)
    t4 = tp.reshape(EMB, H1)
    g = _sc_gather(xf, t4)
    g2 = g.reshape(BATCH, H2)
    return _tc_head(g2, fc_w.T, fc_b.reshape(1, NCLS))


# speedup vs baseline: 2.1556x; 1.0006x over previous
import functools

import jax
import jax.numpy as jnp
from jax import lax
from jax.experimental import pallas as pl
from jax.experimental.pallas import tpu as pltpu
from jax.experimental.pallas import tpu_sc as plsc

EMB = 1_000_000
H1 = 32
HIST = 50
BATCH = 16384
H2 = HIST * H1
NCLS = 3

NC = 2
NS = 16
NW = NC * NS
TOTAL = BATCH * HIST
PER_W = TOTAL // NW
CHUNK = 128
N_CHUNK = PER_W // CHUNK

_mesh = plsc.VectorSubcoreMesh(core_axis_name="c", subcore_axis_name="s")

CT = 8192
NBLK = (EMB + CT - 1) // CT


def _tp_body(in_ref, o_ref):
    t = in_ref[...].T
    q = CT // 4
    for k in range(4):
        o_ref[:, 32 * k:32 * k + 32] = t[k * q:(k + 1) * q, :]


def _tc_transpose_pad(tbl_t):
    return pl.pallas_call(
        _tp_body,
        grid=(NBLK,),
        in_specs=[pl.BlockSpec((H1, CT), lambda j: (0, j))],
        out_specs=pl.BlockSpec((CT // 4, 128), lambda j: (j, 0)),
        out_shape=jax.ShapeDtypeStruct((NBLK * CT // 4, 128), jnp.float32),
    )(tbl_t)


@functools.partial(
    pl.kernel,
    mesh=_mesh,
    out_type=jax.ShapeDtypeStruct((TOTAL, H1), jnp.float32),
    compiler_params=pltpu.CompilerParams(use_tc_tiling_on_sc=False),
    scratch_types=[
        pltpu.VMEM((CHUNK,), jnp.int32),
        pltpu.VMEM((CHUNK, H1), jnp.float32),
        pltpu.SemaphoreType.DMA,
    ],
)
def _sc_gather(idx_hbm, table_hbm, out_hbm, idx_v, rows_v, sem):
    wid = lax.axis_index("s") * NC + lax.axis_index("c")
    base = wid * PER_W

    @pl.loop(0, N_CHUNK)
    def _(i):
        off = base + i * CHUNK
        pltpu.sync_copy(idx_hbm.at[pl.ds(off, CHUNK)], idx_v)
        pltpu.async_copy(table_hbm.at[idx_v], rows_v, sem).wait()
        pltpu.sync_copy(rows_v, out_hbm.at[pl.ds(off, CHUNK)])


def _mm_body(g_ref, w_ref, b_ref, o_ref):
    logits = jnp.dot(g_ref[...], w_ref[...],
                     preferred_element_type=jnp.float32) + b_ref[...]
    m = jnp.max(logits, axis=1, keepdims=True)
    s = logits - m
    lse = jnp.log(jnp.sum(jnp.exp(s), axis=1, keepdims=True))
    o_ref[...] = s - lse


BB = 1024


def _tc_head(g, wt, b2):
    return pl.pallas_call(
        _mm_body,
        grid=(BATCH // BB,),
        in_specs=[
            pl.BlockSpec((BB, H2), lambda i: (i, 0)),
            pl.BlockSpec((H2, NCLS), lambda i: (0, 0)),
            pl.BlockSpec((1, NCLS), lambda i: (0, 0)),
        ],
        out_specs=pl.BlockSpec((BB, NCLS), lambda i: (i, 0)),
        out_shape=jax.ShapeDtypeStruct((BATCH, NCLS), jnp.float32),
    )(g, wt, b2)


def kernel(x, emb_table, fc_w, fc_b):
    xi = x.reshape(-1).astype(jnp.int32)
    xf = (xi & ~(CT - 1)) | ((xi & (CT // 4 - 1)) << 2) | ((xi >> 11) & 3)
    tp = _tc_transpose_pad(emb_table.T)
    t4 = tp.reshape(NBLK * CT, H1)
    g = _sc_gather(xf, t4)
    g2 = g.reshape(BATCH, H2)
    return _tc_head(g2, fc_w.T, fc_b.reshape(1, NCLS))


# speedup vs baseline: 3.0008x; 1.3921x over previous
import functools

import jax
import jax.numpy as jnp
from jax import lax
from jax.experimental import pallas as pl
from jax.experimental.pallas import tpu as pltpu
from jax.experimental.pallas import tpu_sc as plsc

EMB = 1_000_000
H1 = 32
HIST = 50
BATCH = 16384
H2 = HIST * H1
NCLS = 3

NC = 2
NS = 16
NW = NC * NS
TOTAL = BATCH * HIST
PER_W = TOTAL // NW
CHUNK = 128
N_CHUNK = PER_W // CHUNK
QD = 4

_mesh = plsc.VectorSubcoreMesh(core_axis_name="c", subcore_axis_name="s")

CT = 8192
NBLK = (EMB + CT - 1) // CT


def _tp_body(in_ref, o_ref):
    t = in_ref[...].T
    q = CT // 4
    for k in range(4):
        o_ref[:, 32 * k:32 * k + 32] = t[k * q:(k + 1) * q, :]


def _tc_transpose_pad(tbl_t):
    return pl.pallas_call(
        _tp_body,
        grid=(NBLK,),
        in_specs=[pl.BlockSpec((H1, CT), lambda j: (0, j))],
        out_specs=pl.BlockSpec((CT // 4, 128), lambda j: (j, 0)),
        out_shape=jax.ShapeDtypeStruct((NBLK * CT // 4, 128), jnp.float32),
    )(tbl_t)


@functools.partial(
    pl.kernel,
    mesh=_mesh,
    out_type=jax.ShapeDtypeStruct((TOTAL, H1), jnp.float32),
    compiler_params=pltpu.CompilerParams(use_tc_tiling_on_sc=False),
    scratch_types=[
        pltpu.VMEM((PER_W,), jnp.int32),
        pltpu.VMEM((QD, CHUNK, H1), jnp.float32),
        pltpu.SemaphoreType.DMA,
        pltpu.SemaphoreType.DMA((QD,)),
        pltpu.SemaphoreType.DMA((QD,)),
    ],
)
def _sc_gather(idx_hbm, table_hbm, out_hbm, idx_v, rows_v, sem_i, sem_g, sem_s):
    wid = lax.axis_index("s") * NC + lax.axis_index("c")
    base = wid * PER_W
    pltpu.async_copy(idx_hbm.at[pl.ds(base, PER_W)], idx_v, sem_i).wait()

    @pl.loop(0, N_CHUNK, step=QD)
    def _(c):
        gs = []
        for b in range(QD):
            gs.append(pltpu.async_copy(
                table_hbm.at[idx_v.at[pl.ds((c + b) * CHUNK, CHUNK)]],
                rows_v.at[b], sem_g.at[b]))
        ss = []
        for b in range(QD):
            gs[b].wait()
            ss.append(pltpu.async_copy(
                rows_v.at[b],
                out_hbm.at[pl.ds(base + (c + b) * CHUNK, CHUNK)],
                sem_s.at[b]))
        for b in range(QD):
            ss[b].wait()


def _mm_body(g_ref, w_ref, b_ref, o_ref):
    logits = jnp.dot(g_ref[...], w_ref[...],
                     preferred_element_type=jnp.float32) + b_ref[...]
    m = jnp.max(logits, axis=1, keepdims=True)
    s = logits - m
    lse = jnp.log(jnp.sum(jnp.exp(s), axis=1, keepdims=True))
    o_ref[...] = s - lse


BB = 1024


def _tc_head(g, wt, b2):
    return pl.pallas_call(
        _mm_body,
        grid=(BATCH // BB,),
        in_specs=[
            pl.BlockSpec((BB, H2), lambda i: (i, 0)),
            pl.BlockSpec((H2, NCLS), lambda i: (0, 0)),
            pl.BlockSpec((1, NCLS), lambda i: (0, 0)),
        ],
        out_specs=pl.BlockSpec((BB, NCLS), lambda i: (i, 0)),
        out_shape=jax.ShapeDtypeStruct((BATCH, NCLS), jnp.float32),
    )(g, wt, b2)


def kernel(x, emb_table, fc_w, fc_b):
    xi = x.reshape(-1).astype(jnp.int32)
    xf = (xi & ~(CT - 1)) | ((xi & (CT // 4 - 1)) << 2) | ((xi >> 11) & 3)
    tp = _tc_transpose_pad(emb_table.T)
    t4 = tp.reshape(NBLK * CT, H1)
    g = _sc_gather(xf, t4)
    g2 = g.reshape(BATCH, H2)
    return _tc_head(g2, fc_w.T, fc_b.reshape(1, NCLS))


# speedup vs baseline: 3.1040x; 1.0344x over previous
import functools

import jax
import jax.numpy as jnp
from jax import lax
from jax.experimental import pallas as pl
from jax.experimental.pallas import tpu as pltpu
from jax.experimental.pallas import tpu_sc as plsc

EMB = 1_000_000
H1 = 32
HIST = 50
BATCH = 16384
H2 = HIST * H1
NCLS = 3

NC = 2
NS = 16
NW = NC * NS
TOTAL = BATCH * HIST
NSPLIT = 2
TOT_S = TOTAL // NSPLIT
PER_W = TOT_S // NW
CHUNK = 128
N_CHUNK = PER_W // CHUNK
QD = 4

_mesh = plsc.VectorSubcoreMesh(core_axis_name="c", subcore_axis_name="s")

CT = 8192
NBLK = (EMB + CT - 1) // CT


def _tp_body(in_ref, o_ref):
    q = CT // 4
    for k in range(4):
        o_ref[:, 32 * k:32 * k + 32] = in_ref[:, k * q:(k + 1) * q].T


def _tc_transpose_pad(tbl_t):
    return pl.pallas_call(
        _tp_body,
        grid=(NBLK,),
        in_specs=[pl.BlockSpec((H1, CT), lambda j: (0, j))],
        out_specs=pl.BlockSpec((CT // 4, 128), lambda j: (j, 0)),
        out_shape=jax.ShapeDtypeStruct((NBLK * CT // 4, 128), jnp.float32),
    )(tbl_t)


@functools.partial(
    pl.kernel,
    mesh=_mesh,
    out_type=jax.ShapeDtypeStruct((TOT_S, H1), jnp.float32),
    compiler_params=pltpu.CompilerParams(use_tc_tiling_on_sc=False),
    scratch_types=[
        pltpu.VMEM((PER_W,), jnp.int32),
        pltpu.VMEM((QD, CHUNK, H1), jnp.float32),
        pltpu.SemaphoreType.DMA,
        pltpu.SemaphoreType.DMA((QD,)),
        pltpu.SemaphoreType.DMA((QD,)),
    ],
)
def _sc_gather(idx_hbm, table_hbm, out_hbm, idx_v, rows_v, sem_i, sem_g, sem_s):
    wid = lax.axis_index("s") * NC + lax.axis_index("c")
    base = wid * PER_W
    pltpu.async_copy(idx_hbm.at[pl.ds(base, PER_W)], idx_v, sem_i).wait()

    @pl.loop(0, N_CHUNK, step=QD)
    def _(c):
        gs = []
        for b in range(QD):
            gs.append(pltpu.async_copy(
                table_hbm.at[idx_v.at[pl.ds((c + b) * CHUNK, CHUNK)]],
                rows_v.at[b], sem_g.at[b]))
        ss = []
        for b in range(QD):
            gs[b].wait()
            ss.append(pltpu.async_copy(
                rows_v.at[b],
                out_hbm.at[pl.ds(base + (c + b) * CHUNK, CHUNK)],
                sem_s.at[b]))
        for b in range(QD):
            ss[b].wait()


def _mm_body(g_ref, w_ref, b_ref, o_ref):
    logits = jnp.dot(g_ref[...], w_ref[...],
                     preferred_element_type=jnp.float32) + b_ref[...]
    m = jnp.max(logits, axis=1, keepdims=True)
    s = logits - m
    lse = jnp.log(jnp.sum(jnp.exp(s), axis=1, keepdims=True))
    o_ref[...] = s - lse


BB = 1024


def _tc_head(g, wt, b2):
    nb = g.shape[0]
    return pl.pallas_call(
        _mm_body,
        grid=(nb // BB,),
        in_specs=[
            pl.BlockSpec((BB, H2), lambda i: (i, 0)),
            pl.BlockSpec((H2, NCLS), lambda i: (0, 0)),
            pl.BlockSpec((1, NCLS), lambda i: (0, 0)),
        ],
        out_specs=pl.BlockSpec((BB, NCLS), lambda i: (i, 0)),
        out_shape=jax.ShapeDtypeStruct((nb, NCLS), jnp.float32),
    )(g, wt, b2)


def kernel(x, emb_table, fc_w, fc_b):
    xi = x.reshape(-1).astype(jnp.int32)
    xf = (xi & ~(CT - 1)) | ((xi & (CT // 4 - 1)) << 2) | ((xi >> 11) & 3)
    tp = _tc_transpose_pad(emb_table.T)
    t4 = tp.reshape(NBLK * CT, H1)
    wt = fc_w.T
    b2 = fc_b.reshape(1, NCLS)
    outs = []
    for sp in range(NSPLIT):
        gs = _sc_gather(xf[sp * TOT_S:(sp + 1) * TOT_S], t4)
        outs.append(_tc_head(gs.reshape(BATCH // NSPLIT, H2), wt, b2))
    return jnp.concatenate(outs, axis=0)
